# Initial kernel scaffold; baseline (speedup 1.0000x reference)
import jax
import jax.numpy as jnp
from jax.experimental import pallas as pl


def kernel(att_feats, ent_feats_sr, ent_feats_tg, a_w, a_b, W_enc, g1_w, g1_b, g2_w, g2_b, val_feats, trip_sr, trip_tg, adj_sr_row, adj_sr_col, adj_tg_row, adj_tg_col, ent_seed_sr, ent_seed_tg):
    raise NotImplementedError("write your pallas kernel here")


# speedup vs baseline: 8.4742x; 8.4742x over previous
import functools

import jax
import jax.numpy as jnp
from jax import lax
from jax.experimental import pallas as pl
from jax.experimental.pallas import tpu as pltpu
from jax.experimental.pallas import tpu_sc as plsc

F32 = jnp.float32
I32 = jnp.int32

N = 10000
NP = 10240
NST = 2 * NP
DIM = 128
HALF = 64
NV = 50000
ATT_P = 1024
NC, NS = 2, 16
NTILE = NC * NS
RPT = NST // NS
RB = 128

K_ENC = 160
K_GCN = 320
SEED_PT = 288

_mesh = plsc.VectorSubcoreMesh(
    core_axis_name="c", subcore_axis_name="s", num_cores=NC, num_subcores=NS)
_sc_params = pltpu.CompilerParams(needs_layout_passes=False,
                                  use_tc_tiling_on_sc=False)


def _zero_2d(buf, rows):
  def body(r, _):
    for j in range(HALF // 16):
      buf[r, pl.ds(j * 16, 16)] = jnp.zeros((16,), F32)
    return 0
  lax.fori_loop(0, rows, body, 0)


def _zero_1d(buf, n):
  def body(g, _):
    buf[pl.ds(g * 16, 16)] = jnp.zeros((16,), F32)
    return 0
  lax.fori_loop(0, n // 16, body, 0)


def _enc_kernel(e_pad):
  nch = e_pad // (NS * K_ENC)
  assert nch % 2 == 0

  @functools.partial(
      pl.kernel,
      out_type=jax.ShapeDtypeStruct((NC, NST, HALF), F32),
      mesh=_mesh,
      compiler_params=_sc_params,
      scratch_types=[
          pltpu.VMEM((2 * ATT_P,), F32),
          pltpu.VMEM((2, 3, K_ENC), I32),
          pltpu.VMEM((2, K_ENC), F32),
          pltpu.VMEM((2, K_ENC, HALF), F32),
          pltpu.VMEM((2, K_ENC, HALF), F32),
          pltpu.VMEM((2, K_ENC), F32),
          pltpu.VMEM_SHARED((NST, HALF), F32),
          pltpu.VMEM_SHARED((NST,), F32),
      ] + [pltpu.SemaphoreType.DMA] * 10)
  def k(eidx_hbm, se_hbm, sa_hbm, ap_hbm, vp_hbm, ent_hbm,
        out_hbm, sa_v, ib, seb, ga, gv, scb, acc, rs,
        sem_s0, sem_s1, sem_a0, sem_a1, sem_v0, sem_v1,
        sem_w0, sem_w1, sem_r0, sem_r1):
    c = lax.axis_index("c")
    s = lax.axis_index("s")
    ew = e_pad // NS
    sems = ((sem_s0, sem_a0, sem_v0, sem_w0, sem_r0),
            (sem_s1, sem_a1, sem_v1, sem_w1, sem_r1))

    _zero_2d(ga.at[0], RB)
    _zero_1d(scb.at[0], K_ENC)
    def zblk(b, _):
      r0 = s * RPT + b * RB
      pltpu.sync_copy(ga.at[0, pl.ds(0, RB)], acc.at[pl.ds(r0, RB)])
      pltpu.sync_copy(scb.at[0, pl.ds(0, RB)], rs.at[pl.ds(r0, RB)])
      return 0
    lax.fori_loop(0, RPT // RB, zblk, 0)

    pltpu.sync_copy(sa_hbm, sa_v)
    plsc.subcore_barrier()

    def wpair(slot):
      return (pltpu.make_async_copy(ga.at[slot], acc.at[ib.at[slot, 0]],
                                    sems[slot][3]),
              pltpu.make_async_copy(scb.at[slot], rs.at[ib.at[slot, 0]],
                                    sems[slot][4]))

    def stage(slot, i):
      @pl.when(i >= 2)
      def _():
        wa, wr = wpair(slot)
        wa.wait()
        wr.wait()
      base = s * ew + i * K_ENC
      pltpu.sync_copy(eidx_hbm.at[c, :, pl.ds(base, K_ENC)], ib.at[slot])
      pltpu.async_copy(se_hbm.at[ib.at[slot, 0]], seb.at[slot],
                       sems[slot][0])
      pltpu.async_copy(ap_hbm.at[ib.at[slot, 1]], ga.at[slot], sems[slot][1])
      pltpu.async_copy(vp_hbm.at[ib.at[slot, 2]], gv.at[slot], sems[slot][2])

    def consume(slot):
      pltpu.make_async_copy(se_hbm.at[ib.at[slot, 0]], seb.at[slot],
                            sems[slot][0]).wait()
      def sc16(g, _):
        sl = pl.ds(g * 16, 16)
        a16 = ib[slot, 1, sl]
        x = seb[slot, sl] + plsc.load_gather(sa_v, [a16])
        scb[slot, sl] = jnp.exp(jnp.maximum(x, 0.2 * x))
        return 0
      lax.fori_loop(0, K_ENC // 16, sc16, 0)
      pltpu.make_async_copy(ap_hbm.at[ib.at[slot, 1]], ga.at[slot],
                            sems[slot][1]).wait()
      pltpu.make_async_copy(vp_hbm.at[ib.at[slot, 2]], gv.at[slot],
                            sems[slot][2]).wait()
      def scale(g, _):
        for u in range(2):
          e = g * 2 + u
          s16 = plsc.load_gather(scb.at[slot], [jnp.full((16,), e, I32)])
          for j in range(HALF // 16):
            sl = pl.ds(j * 16, 16)
            ga[slot, e, sl] = (ga[slot, e, sl] + gv[slot, e, sl]) * s16
        return 0
      lax.fori_loop(0, K_ENC // 2, scale, 0)
      pltpu.async_copy(ga.at[slot], acc.at[ib.at[slot, 0]], sems[slot][3],
                       add=True)
      pltpu.async_copy(scb.at[slot], rs.at[ib.at[slot, 0]], sems[slot][4],
                       add=True)

    stage(0, 0)
    def pair(p, _):
      i = p * 2
      stage(1, i + 1)
      consume(0)
      @pl.when(i + 2 < nch)
      def _():
        stage(0, i + 2)
      consume(1)
      return 0
    lax.fori_loop(0, nch // 2, pair, 0)
    for slot in range(2):
      wa, wr = wpair(slot)
      wa.wait()
      wr.wait()

    plsc.subcore_barrier()

    fb, eb, rsb = ga.at[0], gv.at[0], scb.at[0]
    def fin(b, _):
      r0 = s * RPT + b * RB
      pltpu.sync_copy(acc.at[pl.ds(r0, RB)], fb.at[pl.ds(0, RB)])
      pltpu.sync_copy(rs.at[pl.ds(r0, RB)], rsb.at[pl.ds(0, RB)])
      pltpu.sync_copy(ent_hbm.at[c, pl.ds(r0, RB)], eb.at[pl.ds(0, RB)])
      def recip(g, _):
        sl = pl.ds(g * 16, 16)
        rsb[sl] = 1.0 / rsb[sl]
        return 0
      lax.fori_loop(0, RB // 16, recip, 0)
      def rows(r, _):
        s16 = plsc.load_gather(rsb, [jnp.full((16,), r, I32)])
        for j in range(HALF // 16):
          sl = pl.ds(j * 16, 16)
          x = fb[r, sl] * s16 + eb[r, sl]
          fb[r, sl] = jnp.where(x > 0.0, x, jnp.exp(x) - 1.0)
        return 0
      lax.fori_loop(0, RB, rows, 0)
      pltpu.sync_copy(fb.at[pl.ds(0, RB)], out_hbm.at[c, pl.ds(r0, RB)])
      return 0
    lax.fori_loop(0, RPT // RB, fin, 0)

  return k


def _gcn_kernel(e_pad):
  nch = e_pad // (NS * K_GCN)
  assert nch % 2 == 0

  @functools.partial(
      pl.kernel,
      out_type=[jax.ShapeDtypeStruct((NC, NST, HALF), F32),
                jax.ShapeDtypeStruct((NST,), F32)],
      mesh=_mesh,
      compiler_params=_sc_params,
      scratch_types=[
          pltpu.VMEM((2, 2, K_GCN), I32),
          pltpu.VMEM((2, K_GCN, HALF), F32),
          pltpu.VMEM((K_GCN,), F32),
          pltpu.VMEM_SHARED((NST, HALF), F32),
          pltpu.VMEM_SHARED((NST,), F32),
      ] + [pltpu.SemaphoreType.DMA] * 6)
  def k(gidx_hbm, feats_hbm, agg_hbm, deg_hbm,
        ib, g, ones, acc, deg, sem_g0, sem_g1, sem_w0, sem_w1,
        sem_d0, sem_d1):
    c = lax.axis_index("c")
    s = lax.axis_index("s")
    ew = e_pad // NS
    sems = ((sem_g0, sem_w0, sem_d0), (sem_g1, sem_w1, sem_d1))

    _zero_2d(g.at[0], RB)
    _zero_1d(ones, K_GCN)
    def zblk(b, _):
      r0 = s * RPT + b * RB
      pltpu.sync_copy(g.at[0, pl.ds(0, RB)], acc.at[pl.ds(r0, RB)])
      pltpu.sync_copy(ones.at[pl.ds(0, RB)], deg.at[pl.ds(r0, RB)])
      return 0
    lax.fori_loop(0, RPT // RB, zblk, 0)
    def ob(gi, _):
      ones[pl.ds(gi * 16, 16)] = jnp.full((16,), 1.0, F32)
      return 0
    lax.fori_loop(0, K_GCN // 16, ob, 0)
    plsc.subcore_barrier()

    def wpair(slot):
      return (pltpu.make_async_copy(g.at[slot], acc.at[ib.at[slot, 1]],
                                    sems[slot][1]),
              pltpu.make_async_copy(ones, deg.at[ib.at[slot, 1]],
                                    sems[slot][2]))

    def stage(slot, i):
      @pl.when(i >= 2)
      def _():
        wa, wd = wpair(slot)
        wa.wait()
        @pl.when(c == 0)
        def _():
          wd.wait()
      base = s * ew + i * K_GCN
      pltpu.sync_copy(gidx_hbm.at[c, :, pl.ds(base, K_GCN)], ib.at[slot])
      pltpu.async_copy(feats_hbm.at[ib.at[slot, 0]], g.at[slot],
                       sems[slot][0])

    def consume(slot):
      pltpu.make_async_copy(feats_hbm.at[ib.at[slot, 0]], g.at[slot],
                            sems[slot][0]).wait()
      pltpu.async_copy(g.at[slot], acc.at[ib.at[slot, 1]], sems[slot][1],
                       add=True)
      @pl.when(c == 0)
      def _():
        pltpu.async_copy(ones, deg.at[ib.at[slot, 1]], sems[slot][2],
                         add=True)

    stage(0, 0)
    def pair(p, _):
      i = p * 2
      stage(1, i + 1)
      consume(0)
      @pl.when(i + 2 < nch)
      def _():
        stage(0, i + 2)
      consume(1)
      return 0
    lax.fori_loop(0, nch // 2, pair, 0)
    for slot in range(2):
      wa, wd = wpair(slot)
      wa.wait()
      @pl.when(c == 0)
      def _():
        wd.wait()

    plsc.subcore_barrier()

    def dump(b, _):
      r0 = s * RPT + b * RB
      pltpu.sync_copy(acc.at[pl.ds(r0, RB)], g.at[0, pl.ds(0, RB)])
      pltpu.sync_copy(g.at[0, pl.ds(0, RB)], agg_hbm.at[c, pl.ds(r0, RB)])
      return 0
    lax.fori_loop(0, RPT // RB, dump, 0)

    @pl.when(c == 0)
    def _():
      def dump1(b, _):
        r0 = s * RPT + b * RB
        pltpu.sync_copy(deg.at[pl.ds(r0, RB)], ones.at[pl.ds(0, RB)])
        pltpu.sync_copy(ones.at[pl.ds(0, RB)], deg_hbm.at[pl.ds(r0, RB)])
        return 0
      lax.fori_loop(0, RPT // RB, dump1, 0)

  return k


@functools.partial(
    pl.kernel,
    out_type=jax.ShapeDtypeStruct((NTILE * SEED_PT, DIM), F32),
    mesh=_mesh,
    compiler_params=_sc_params,
    scratch_types=[
        pltpu.VMEM((SEED_PT,), I32),
        pltpu.VMEM((SEED_PT, DIM), F32),
        pltpu.SemaphoreType.DMA,
    ])
def _seed_kernel(fin_hbm, seed_hbm, out_hbm, ib, g, sem):
  c = lax.axis_index("c")
  s = lax.axis_index("s")
  wid = s * NC + c
  base = wid * SEED_PT
  pltpu.sync_copy(seed_hbm.at[pl.ds(base, SEED_PT)], ib)
  pltpu.async_copy(fin_hbm.at[ib], g, sem).wait()
  pltpu.sync_copy(g, out_hbm.at[pl.ds(base, SEED_PT)])


def _valproj_body(val_ref, w_ref, out_ref):
  vp = jnp.dot(val_ref[...], w_ref[...], preferred_element_type=F32)
  out_ref[0] = vp[:, :HALF]
  out_ref[1] = vp[:, HALF:]


def _val_proj(val_feats, w_r):
  nb = 25
  bs = NV // nb
  return pl.pallas_call(
      _valproj_body,
      grid=(nb,),
      in_specs=[pl.BlockSpec((bs, DIM), lambda i: (i, 0)),
                pl.BlockSpec((DIM, DIM), lambda i: (0, 0))],
      out_specs=pl.BlockSpec((NC, bs, HALF), lambda i: (0, i, 0)),
      out_shape=jax.ShapeDtypeStruct((NC, NV, HALF), F32),
  )(val_feats, w_r)


def _pre_body(att_ref, ent_ref, aw_ref, ab_ref, wl_ref, ap_ref, sa_ref, se_ref):
  aw = aw_ref[...]
  ap = jnp.dot(att_ref[...], wl_ref[...], preferred_element_type=F32)
  ap_ref[0] = ap[:, :HALF]
  ap_ref[1] = ap[:, HALF:]
  awr8 = jnp.concatenate([aw[:, DIM:].T, jnp.zeros((DIM, 7), F32)], axis=1)
  awl8 = jnp.concatenate([aw[:, :DIM].T, jnp.zeros((DIM, 7), F32)], axis=1)
  sa_ref[...] = jnp.dot(att_ref[...], awr8,
                        preferred_element_type=F32)[:, :1]
  se_ref[...] = jnp.dot(ent_ref[...], awl8,
                        preferred_element_type=F32)[:, :1] + ab_ref[0, 0]


def _precompute(att_p, ent_full, a_w, a_b, w_l):
  return pl.pallas_call(
      _pre_body,
      in_specs=[pl.BlockSpec((ATT_P, DIM), lambda: (0, 0)),
                pl.BlockSpec((NST, DIM), lambda: (0, 0)),
                pl.BlockSpec((1, 2 * DIM), lambda: (0, 0)),
                pl.BlockSpec((1, 1), lambda: (0, 0)),
                pl.BlockSpec((DIM, DIM), lambda: (0, 0))],
      out_specs=[pl.BlockSpec((NC, ATT_P, HALF), lambda: (0, 0, 0)),
                 pl.BlockSpec((ATT_P, 1), lambda: (0, 0)),
                 pl.BlockSpec((NST, 1), lambda: (0, 0))],
      out_shape=[jax.ShapeDtypeStruct((NC, ATT_P, HALF), F32),
                 jax.ShapeDtypeStruct((ATT_P, 1), F32),
                 jax.ShapeDtypeStruct((NST, 1), F32)],
  )(att_p, ent_full, a_w, a_b, w_l)


def _mm_body(relu, norm, agg_ref, feats_ref, deg_ref, w_ref, b_ref, out_ref):
  w = w_ref[...]
  t = (lax.dot_general(agg_ref[0], w[:, :HALF], (((1,), (1,)), ((), ())),
                       preferred_element_type=F32) +
       lax.dot_general(agg_ref[1], w[:, HALF:], (((1,), (1,)), ((), ())),
                       preferred_element_type=F32))
  t = t / deg_ref[...] + b_ref[...]
  if relu:
    t = jnp.maximum(t, 0.0)
  y = jnp.concatenate([feats_ref[0], feats_ref[1]], axis=1) + t
  if norm:
    n = jnp.sqrt(jnp.sum(y * y, axis=1, keepdims=True))
    y = y / jnp.maximum(n, 1e-12)
    out_ref[...] = y
  else:
    out_ref[0] = y[:, :HALF]
    out_ref[1] = y[:, HALF:]


def _gcn_dense(agg, feats, deg, w, b, relu, norm):
  nb = 10
  bs = NST // nb
  if norm:
    out_specs = pl.BlockSpec((bs, DIM), lambda i: (i, 0))
    out_shape = jax.ShapeDtypeStruct((NST, DIM), F32)
  else:
    out_specs = pl.BlockSpec((NC, bs, HALF), lambda i: (0, i, 0))
    out_shape = jax.ShapeDtypeStruct((NC, NST, HALF), F32)
  return pl.pallas_call(
      functools.partial(_mm_body, relu, norm),
      grid=(nb,),
      in_specs=[pl.BlockSpec((NC, bs, HALF), lambda i: (0, i, 0)),
                pl.BlockSpec((NC, bs, HALF), lambda i: (0, i, 0)),
                pl.BlockSpec((bs, 1), lambda i: (i, 0)),
                pl.BlockSpec((DIM, DIM), lambda i: (0, 0)),
                pl.BlockSpec((1, DIM), lambda i: (0, 0))],
      out_specs=out_specs,
      out_shape=out_shape,
  )(agg, feats, deg, w, b)


def _pad_idx(parts, total, fill_fn):
  cur = sum(p.shape[0] for p in parts)
  padn = total - cur
  if padn:
    parts = list(parts) + [fill_fn(padn)]
  return jnp.concatenate(parts)


def kernel(att_feats, ent_feats_sr, ent_feats_tg, a_w, a_b, W_enc,
           g1_w, g1_b, g2_w, g2_b, val_feats, trip_sr, trip_tg,
           adj_sr_row, adj_sr_col, adj_tg_row, adj_tg_col,
           ent_seed_sr, ent_seed_tg):
  zpad = jnp.zeros((NP - N, DIM), F32)
  ent_full = jnp.concatenate([ent_feats_sr, zpad, ent_feats_tg, zpad])
  ent_h = jnp.stack([ent_full[:, :HALF], ent_full[:, HALF:]])
  att_p = jnp.concatenate(
      [att_feats, jnp.zeros((ATT_P - att_feats.shape[0], DIM), F32)])

  vp2 = _val_proj(val_feats, W_enc[DIM:]).reshape(NC * NV, HALF)
  ap3, sa2, se2 = _precompute(att_p, ent_full, a_w,
                              a_b.reshape(1, 1), W_enc[:DIM])
  ap2 = ap3.reshape(NC * ATT_P, HALF)
  sa = sa2.reshape(ATT_P)
  sa_c = jnp.concatenate([sa, sa])
  se = se2.reshape(NST)

  e_att = trip_sr.shape[0] + trip_tg.shape[0]
  e_att_pad = -(-e_att // (2 * NS * K_ENC)) * (2 * NS * K_ENC)
  garbage = lambda n: (N + (jnp.arange(n, dtype=I32) % (NP - N))).astype(I32)
  h_all = _pad_idx([trip_sr[:, 0], trip_tg[:, 0] + NP], e_att_pad, garbage)
  att_all = _pad_idx([trip_sr[:, 2], trip_tg[:, 2]], e_att_pad,
                     lambda n: jnp.arange(n, dtype=I32) % 1001)
  val_all = _pad_idx([trip_sr[:, 1], trip_tg[:, 1]], e_att_pad,
                     lambda n: jnp.arange(n, dtype=I32) % NV)
  eidx = jnp.stack([
      jnp.stack([h_all, att_all, val_all]),
      jnp.stack([h_all, att_all + ATT_P, val_all + NV])])

  e_g = adj_sr_row.shape[0] + adj_tg_row.shape[0]
  e_g_pad = -(-e_g // (2 * NS * K_GCN)) * (2 * NS * K_GCN)
  col_all = _pad_idx([adj_sr_col, adj_tg_col + NP], e_g_pad,
                     lambda n: jnp.arange(n, dtype=I32) % 4096)
  row_all = _pad_idx([adj_sr_row, adj_tg_row + NP], e_g_pad, garbage)
  gidx = jnp.stack([
      jnp.stack([col_all, row_all]),
      jnp.stack([col_all + NST, row_all])])

  enc3 = _enc_kernel(e_att_pad)(eidx, se, sa_c, ap2, vp2, ent_h)
  enc2 = enc3.reshape(NC * NST, HALF)

  gk = _gcn_kernel(e_g_pad)
  agg1, deg = gk(gidx, enc2)
  deg2 = deg.reshape(NST, 1)
  out1 = _gcn_dense(agg1, enc3, deg2, g1_w, g1_b.reshape(1, DIM),
                    relu=True, norm=False)

  agg2, _ = gk(gidx, out1.reshape(NC * NST, HALF))
  fin = _gcn_dense(agg2, out1, deg2, g2_w, g2_b.reshape(1, DIM),
                   relu=False, norm=True)

  ns = ent_seed_sr.shape[0]
  spt = NTILE * SEED_PT // 2
  seeds = _pad_idx([ent_seed_sr], spt, lambda n: jnp.zeros((n,), I32))
  seedt = _pad_idx([ent_seed_tg + NP], NTILE * SEED_PT - spt,
                   lambda n: jnp.zeros((n,), I32))
  gathered = _seed_kernel(fin, jnp.concatenate([seeds, seedt]))

  fsr = fin[:N]
  ftg = fin[NP:NP + N]
  return (gathered[:ns], gathered[spt:spt + ns], fsr, ftg)
